# baseline (speedup 1.0000x reference)
import jax
import jax.numpy as jnp
from jax.experimental import pallas as pl
from jax.experimental.pallas import tpu as pltpu

_H = 200
_G = 2


def _dot(a, b):
    return jnp.dot(a, b, preferred_element_type=jnp.float32)


def _dot_nt(a, b):
    return jax.lax.dot_general(a, b, (((1,), (1,)), ((), ())),
                               preferred_element_type=jnp.float32)


def _gru(x, h, ws):
    wir, wiz, win, whr, whz, whn, br, bz, bni, bnh = ws
    xb = x.astype(jnp.bfloat16)
    r = jax.nn.sigmoid(_dot_nt(xb, wir) + _dot_nt(h, whr) + br)
    z = jax.nn.sigmoid(_dot_nt(xb, wiz) + _dot_nt(h, whz) + bz)
    n = jnp.tanh(_dot_nt(xb, win) + bni + r * (_dot_nt(h, whn) + bnh))
    return (n + z * (h.astype(jnp.float32) - n)).astype(jnp.bfloat16)


def _encode_kernel(adj_ref, gin_ref, ke_ref, wi0_ref, wi5_ref, wh6_ref,
                   bi_ref, bh_ref, wm_ref, bm_ref, out_ref):
    step = pl.program_id(0)
    n = adj_ref.shape[2]

    def wset(s):
        if s == 0:
            wi = (wi0_ref[0], wi0_ref[1], wi0_ref[2])
        else:
            wi = (wi5_ref[s - 1, 0], wi5_ref[s - 1, 1], wi5_ref[s - 1, 2])
        wh = (wh6_ref[s, 0], wh6_ref[s, 1], wh6_ref[s, 2])
        br = bi_ref[3 * s:3 * s + 1] + bh_ref[3 * s:3 * s + 1]
        bz = bi_ref[3 * s + 1:3 * s + 2] + bh_ref[3 * s + 1:3 * s + 2]
        bni = bi_ref[3 * s + 2:3 * s + 3]
        bnh = bh_ref[3 * s + 2:3 * s + 3]
        return wi + wh + (br, bz, bni, bnh)

    fw0, fw1, fw2, bw0, bw1, var = [wset(s) for s in range(6)]

    An, degs, invouts = [], [], []
    for g in range(_G):
        Af = (adj_ref[0, g] < (16.0 / n)).astype(jnp.float32)
        deg_in = jnp.clip(jnp.sum(Af, axis=1, keepdims=True), 1.0, None)
        inv_out = (1.0 / jnp.clip(jnp.sum(Af, axis=0, keepdims=True), 1.0,
                                  None)).T
        An.append((Af * (1.0 / deg_in)).astype(jnp.bfloat16))
        degs.append(deg_in)
        invouts.append(inv_out)

    gin = gin_ref[0].astype(jnp.bfloat16)

    def fwd_msg(feat):
        return jnp.concatenate(
            [_dot(An[g], feat[g * n:(g + 1) * n]) for g in range(_G)], axis=0)

    def bwd_msg(h):
        outs = []
        for g in range(_G):
            hg = (h[g * n:(g + 1) * n] * degs[g].astype(jnp.bfloat16))
            mg = jax.lax.dot_general(
                An[g], hg, (((0,), (0,)), ((), ())),
                preferred_element_type=jnp.float32)
            outs.append(mg * invouts[g])
        return jnp.concatenate(outs, axis=0)

    h = jnp.zeros((_G * n, _H), dtype=jnp.bfloat16)

    def snap(hcur):
        rows = [hcur[g * n + k: g * n + k + 1]
                for k in range(3) for g in range(_G)]
        rows.append(jnp.zeros((16 - 3 * _G, _H), dtype=jnp.bfloat16))
        return jnp.concatenate(rows, axis=0)

    h = _gru(fwd_msg(gin), h, fw0)
    out0 = snap(h)
    h = _gru(bwd_msg(h), h, bw0)
    h = _gru(fwd_msg(h), h, fw1)
    out1 = snap(h)
    h = _gru(bwd_msg(h), h, bw1)
    h = _gru(fwd_msg(h), h, fw2)
    out2 = snap(h)

    hv = jnp.zeros((16, _H), dtype=jnp.bfloat16)
    hv = _gru(out0, hv, var)
    hv = _gru(out1, hv, var)
    hv = _gru(out2, hv, var)

    hvb = hv
    enc = (_dot_nt(hvb[0:_G], wm_ref[0])
           + _dot_nt(hvb[_G:2 * _G], wm_ref[1])
           + _dot_nt(hvb[2 * _G:3 * _G], wm_ref[2])
           + bm_ref[...])
    diff = enc - ke_ref[0]
    del step
    out_ref[0] = jnp.sum(diff * diff, keepdims=True).reshape(1, 1)


def kernel(g_in, g_adj, batch_size, kernel_embeddings, reg_solutions, params):
    B, N, VT = g_in.shape
    Z = kernel_embeddings.shape[1]
    steps = B // _G
    p = params

    wi0 = p["fw"][0]["Wi"].reshape(3, _H, VT).astype(jnp.bfloat16)
    wi5 = (jnp.stack([p["fw"][1]["Wi"], p["fw"][2]["Wi"], p["bw"][0]["Wi"],
                      p["bw"][1]["Wi"], p["var"]["Wi"]])
           .reshape(5, 3, _H, _H).astype(jnp.bfloat16))
    wh6 = (jnp.stack([p["fw"][0]["Wh"], p["fw"][1]["Wh"], p["fw"][2]["Wh"],
                      p["bw"][0]["Wh"], p["bw"][1]["Wh"], p["var"]["Wh"]])
           .reshape(6, 3, _H, _H).astype(jnp.bfloat16))
    bi18 = jnp.stack([p["fw"][0]["bi"], p["fw"][1]["bi"], p["fw"][2]["bi"],
                      p["bw"][0]["bi"], p["bw"][1]["bi"], p["var"]["bi"]]
                     ).reshape(18, _H)
    bh18 = jnp.stack([p["fw"][0]["bh"], p["fw"][1]["bh"], p["fw"][2]["bh"],
                      p["bw"][0]["bh"], p["bw"][1]["bh"], p["var"]["bh"]]
                     ).reshape(18, _H)
    wm3 = p["Wm"].reshape(Z, 3, _H).transpose(1, 0, 2).astype(jnp.bfloat16)
    bm = p["bm"].reshape(1, -1)

    adj4 = g_adj.reshape(steps, _G, N, N)
    gin2 = g_in.reshape(steps, _G * N, VT)
    ke3 = kernel_embeddings.reshape(steps, _G, Z)

    const = lambda shape: pl.BlockSpec(shape, lambda s: (0,) * len(shape))
    in_specs = [
        pl.BlockSpec((1, _G, N, N), lambda s: (s, 0, 0, 0)),
        pl.BlockSpec((1, _G * N, VT), lambda s: (s, 0, 0)),
        pl.BlockSpec((1, _G, Z), lambda s: (s, 0, 0)),
        const(wi0.shape), const(wi5.shape), const(wh6.shape),
        const(bi18.shape), const(bh18.shape), const(wm3.shape),
        const(bm.shape),
    ]

    loss = pl.pallas_call(
        _encode_kernel,
        grid=(steps,),
        in_specs=in_specs,
        out_specs=pl.BlockSpec((1, 1, 1), lambda s: (s, 0, 0)),
        out_shape=jax.ShapeDtypeStruct((steps, 1, 1), jnp.float32),
        compiler_params=pltpu.CompilerParams(
            dimension_semantics=("parallel",),
        ),
    )(adj4, gin2, ke3, wi0, wi5, wh6, bi18, bh18, wm3, bm)
    return jnp.sum(loss)


# speedup vs baseline: 1.0506x; 1.0506x over previous
import jax
import jax.numpy as jnp
from jax.experimental import pallas as pl
from jax.experimental.pallas import tpu as pltpu

_H = 200
_G = 4


def _dot(a, b):
    return jnp.dot(a, b, preferred_element_type=jnp.float32)


def _dot_nt(a, b):
    return jax.lax.dot_general(a, b, (((1,), (1,)), ((), ())),
                               preferred_element_type=jnp.float32)


def _gru(x, h, ws):
    wir, wiz, win, whr, whz, whn, br, bz, bni, bnh = ws
    xb = x.astype(jnp.bfloat16)
    r = jax.nn.sigmoid(_dot_nt(xb, wir) + _dot_nt(h, whr) + br)
    z = jax.nn.sigmoid(_dot_nt(xb, wiz) + _dot_nt(h, whz) + bz)
    n = jnp.tanh(_dot_nt(xb, win) + bni + r * (_dot_nt(h, whn) + bnh))
    return (n + z * (h.astype(jnp.float32) - n)).astype(jnp.bfloat16)


def _encode_kernel(adj_ref, gin_ref, ke_ref, wi0_ref, w11_ref,
                   b36_ref, wm_ref, bm_ref, out_ref):
    step = pl.program_id(0)
    n = adj_ref.shape[2]

    def wset(s):
        if s == 0:
            wi = (wi0_ref[0], wi0_ref[1], wi0_ref[2])
        else:
            wi = (w11_ref[s - 1, 0], w11_ref[s - 1, 1], w11_ref[s - 1, 2])
        wh = (w11_ref[5 + s, 0], w11_ref[5 + s, 1], w11_ref[5 + s, 2])
        br = b36_ref[3 * s:3 * s + 1] + b36_ref[18 + 3 * s:19 + 3 * s]
        bz = (b36_ref[3 * s + 1:3 * s + 2]
              + b36_ref[18 + 3 * s + 1:18 + 3 * s + 2])
        bni = b36_ref[3 * s + 2:3 * s + 3]
        bnh = b36_ref[18 + 3 * s + 2:18 + 3 * s + 3]
        return wi + wh + (br, bz, bni, bnh)

    fw0, fw1, fw2, bw0, bw1, var = [wset(s) for s in range(6)]

    An, degs, invouts = [], [], []
    for g in range(_G):
        Af = (adj_ref[0, g] < (16.0 / n)).astype(jnp.float32)
        deg_in = jnp.clip(jnp.sum(Af, axis=1, keepdims=True), 1.0, None)
        inv_out = (1.0 / jnp.clip(jnp.sum(Af, axis=0, keepdims=True), 1.0,
                                  None)).T
        An.append((Af * (1.0 / deg_in)).astype(jnp.bfloat16))
        degs.append(deg_in)
        invouts.append(inv_out)

    gin = gin_ref[0].astype(jnp.bfloat16)

    def fwd_msg(feat):
        return jnp.concatenate(
            [_dot(An[g], feat[g * n:(g + 1) * n]) for g in range(_G)], axis=0)

    def bwd_msg(h):
        outs = []
        for g in range(_G):
            hg = (h[g * n:(g + 1) * n] * degs[g].astype(jnp.bfloat16))
            mg = jax.lax.dot_general(
                An[g], hg, (((0,), (0,)), ((), ())),
                preferred_element_type=jnp.float32)
            outs.append(mg * invouts[g])
        return jnp.concatenate(outs, axis=0)

    h = jnp.zeros((_G * n, _H), dtype=jnp.bfloat16)

    def snap(hcur):
        rows = [hcur[g * n + k: g * n + k + 1]
                for k in range(3) for g in range(_G)]
        rows.append(jnp.zeros((16 - 3 * _G, _H), dtype=jnp.bfloat16))
        return jnp.concatenate(rows, axis=0)

    h = _gru(fwd_msg(gin), h, fw0)
    out0 = snap(h)
    h = _gru(bwd_msg(h), h, bw0)
    h = _gru(fwd_msg(h), h, fw1)
    out1 = snap(h)
    h = _gru(bwd_msg(h), h, bw1)
    h = _gru(fwd_msg(h), h, fw2)
    out2 = snap(h)

    hv = jnp.zeros((16, _H), dtype=jnp.bfloat16)
    hv = _gru(out0, hv, var)
    hv = _gru(out1, hv, var)
    hv = _gru(out2, hv, var)

    hvb = hv
    enc = (_dot_nt(hvb[0:_G], wm_ref[0])
           + _dot_nt(hvb[_G:2 * _G], wm_ref[1])
           + _dot_nt(hvb[2 * _G:3 * _G], wm_ref[2])
           + bm_ref[...])
    diff = enc - ke_ref[0]
    partial = jnp.sum(diff * diff)

    @pl.when(step == 0)
    def _():
        out_ref[...] = jnp.zeros_like(out_ref)

    out_ref[...] += partial


def kernel(g_in, g_adj, batch_size, kernel_embeddings, reg_solutions, params):
    B, N, VT = g_in.shape
    Z = kernel_embeddings.shape[1]
    steps = B // _G
    p = params

    wi0 = p["fw"][0]["Wi"].reshape(3, _H, VT).astype(jnp.bfloat16)
    w11 = (jnp.stack([p["fw"][1]["Wi"], p["fw"][2]["Wi"], p["bw"][0]["Wi"],
                      p["bw"][1]["Wi"], p["var"]["Wi"],
                      p["fw"][0]["Wh"], p["fw"][1]["Wh"], p["fw"][2]["Wh"],
                      p["bw"][0]["Wh"], p["bw"][1]["Wh"], p["var"]["Wh"]])
           .reshape(11, 3, _H, _H).astype(jnp.bfloat16))
    b36 = jnp.stack([p["fw"][0]["bi"], p["fw"][1]["bi"], p["fw"][2]["bi"],
                     p["bw"][0]["bi"], p["bw"][1]["bi"], p["var"]["bi"],
                     p["fw"][0]["bh"], p["fw"][1]["bh"], p["fw"][2]["bh"],
                     p["bw"][0]["bh"], p["bw"][1]["bh"], p["var"]["bh"]]
                    ).reshape(36, _H)
    wm3 = p["Wm"].reshape(Z, 3, _H).transpose(1, 0, 2).astype(jnp.bfloat16)
    bm = p["bm"].reshape(1, -1)

    adj4 = g_adj.reshape(steps, _G, N, N)
    gin2 = g_in.reshape(steps, _G * N, VT)
    ke3 = kernel_embeddings.reshape(steps, _G, Z)

    const = lambda shape: pl.BlockSpec(shape, lambda s: (0,) * len(shape))
    in_specs = [
        pl.BlockSpec((1, _G, N, N), lambda s: (s, 0, 0, 0)),
        pl.BlockSpec((1, _G * N, VT), lambda s: (s, 0, 0)),
        pl.BlockSpec((1, _G, Z), lambda s: (s, 0, 0)),
        const(wi0.shape), const(w11.shape),
        const(b36.shape), const(wm3.shape),
        const(bm.shape),
    ]

    loss = pl.pallas_call(
        _encode_kernel,
        grid=(steps,),
        in_specs=in_specs,
        out_specs=pl.BlockSpec((1, 1), lambda s: (0, 0)),
        out_shape=jax.ShapeDtypeStruct((1, 1), jnp.float32),
        compiler_params=pltpu.CompilerParams(
            dimension_semantics=("arbitrary",),
        ),
    )(adj4, gin2, ke3, wi0, w11, b36, wm3, bm)
    return loss[0, 0]
